# baseline (speedup 1.0000x reference)
import jax
import jax.numpy as jnp
from jax import lax
from jax.experimental import pallas as pl
from jax.experimental.pallas import tpu as pltpu
from jax.experimental.pallas import tpu_sc as plsc

BATCH = 128
NCAT = 100000
RB = 8

SC_ROWS = 40
TC_ROWS = BATCH - SC_ROWS
NRB_H = TC_ROWS // RB
NRB_L = SC_ROWS // RB

CW = 2048
NFULL_H = NCAT // CW
TAIL_H = NCAT - NFULL_H * CW

CWL = 1024
NFULL_L = NCAT // CWL
TAIL_L = NCAT - NFULL_L * CWL

NWORK = 32
CHK = 1600
NCHUNK = SC_ROWS * NCAT // CHK
NITER = -(-NCHUNK // NWORK)
VPG = 10

_K2 = 1234
_KS2 = _K2 ^ 0x1BD11BDA
_ROT0 = (13, 15, 26, 6)
_ROT1 = (17, 29, 16, 24)


def _rotl(x, d):
    return (x << jnp.uint32(d)) | (x >> jnp.uint32(32 - d))


def _threefry_bits(x1):
    x0 = x1
    x1 = _rotl(x1, 13) ^ x0
    for r in _ROT0[1:]:
        x0 = x0 + x1
        x1 = _rotl(x1, r) ^ x0
    x0 = x0 + jnp.uint32(_K2)
    x1 = x1 + jnp.uint32(_KS2 + 1)
    for r in _ROT1:
        x0 = x0 + x1
        x1 = _rotl(x1, r) ^ x0
    x0 = x0 + jnp.uint32(_KS2)
    x1 = x1 + jnp.uint32(2)
    for r in _ROT0:
        x0 = x0 + x1
        x1 = _rotl(x1, r) ^ x0
    x1 = x1 + jnp.uint32(_K2 + 3)
    for r in _ROT1:
        x0 = x0 + x1
        x1 = _rotl(x1, r) ^ x0
    x0 = x0 + jnp.uint32(_K2)
    x1 = x1 + jnp.uint32(_KS2 + 4)
    for r in _ROT0:
        x0 = x0 + x1
        x1 = _rotl(x1, r) ^ x0
    x0 = x0 + jnp.uint32(_KS2)
    x1 = x1 + jnp.uint32(5)
    return x0 ^ x1


def _gumbel_from_bits(bits):
    fb = (bits >> jnp.uint32(9)) | jnp.uint32(0x3F800000)
    floats = lax.bitcast_convert_type(fb, jnp.float32) - jnp.float32(1.0)
    u = jnp.maximum(jnp.float32(1.1754943508222875e-38), floats)
    return -jnp.log(-jnp.log(u))


_SC_CNT_BASE = TC_ROWS * NCAT + _K2


def _sc_bits_body(out_hbm, buf, lane):
    wid = lax.axis_index("s") * 2 + lax.axis_index("c")
    lane[...] = lax.iota(jnp.int32, 16)

    def chunk(t, _):
        c = wid + t * NWORK

        @pl.when(c < NCHUNK)
        def _():
            cnt0 = _SC_CNT_BASE + c * CHK

            def group(i, _):
                for u in range(VPG):
                    off = i * (VPG * 16) + u * 16
                    x1 = (cnt0 + off + lane[...]).astype(jnp.uint32)
                    buf[pl.ds(off, 16)] = lax.bitcast_convert_type(
                        _threefry_bits(x1), jnp.int32)
                return 0

            lax.fori_loop(0, CHK // (VPG * 16), group, 0)
            pltpu.sync_copy(buf, out_hbm.at[pl.ds(c * CHK, CHK)])

        return 0

    lax.fori_loop(0, NITER, chunk, 0)


def _sc_bits():
    kern = pl.kernel(
        _sc_bits_body,
        out_type=jax.ShapeDtypeStruct((SC_ROWS * NCAT,), jnp.int32),
        mesh=plsc.VectorSubcoreMesh(core_axis_name="c", subcore_axis_name="s"),
        scratch_types=[
            pltpu.VMEM((CHK,), jnp.int32),
            pltpu.VMEM((16,), jnp.int32),
        ],
    )
    return kern().reshape(SC_ROWS, NCAT)


def _heavy_body(x_ref, out_ref, idx_scr):
    s = pl.program_id(0)

    @pl.when(s < NRB_H)
    def _argmax():
        row = s * RB + lax.broadcasted_iota(jnp.int32, (RB, CW), 0)
        basep = row * NCAT + jnp.int32(_K2)
        col0 = lax.broadcasted_iota(jnp.int32, (RB, CW), 1)

        def body(j, carry):
            run_z, run_c = carry
            col = j * CW + col0
            x1 = (basep + col).astype(jnp.uint32)
            x = x_ref[:, pl.ds(j * CW, CW)]
            z = x + _gumbel_from_bits(_threefry_bits(x1))
            better = z > run_z
            return (jnp.where(better, z, run_z), jnp.where(better, col, run_c))

        init = (jnp.full((RB, CW), -jnp.inf, jnp.float32),
                jnp.zeros((RB, CW), jnp.int32))
        run_z, run_c = lax.fori_loop(0, NFULL_H, body, init)

        colt = NFULL_H * CW + lax.broadcasted_iota(jnp.int32, (RB, TAIL_H), 1)
        rowt = s * RB + lax.broadcasted_iota(jnp.int32, (RB, TAIL_H), 0)
        x1t = (rowt * NCAT + jnp.int32(_K2) + colt).astype(jnp.uint32)
        xt = x_ref[:, pl.ds(NFULL_H * CW, TAIL_H)]
        zt = xt + _gumbel_from_bits(_threefry_bits(x1t))

        big = jnp.int32(2**31 - 1)
        rmax = jnp.maximum(jnp.max(run_z, axis=1, keepdims=True),
                           jnp.max(zt, axis=1, keepdims=True))
        cand = jnp.min(jnp.where(run_z == rmax, run_c, big),
                       axis=1, keepdims=True)
        candt = jnp.min(jnp.where(zt == rmax, colt, big),
                        axis=1, keepdims=True)
        idx_scr[pl.ds(s * RB, RB), :] = jnp.minimum(cand, candt)

    @pl.when(s > 0)
    def _onehot():
        idx = idx_scr[pl.ds((s - 1) * RB, RB), :]
        col = lax.broadcasted_iota(jnp.int32, (RB, NCAT), 1)
        out_ref[...] = (col == idx).astype(jnp.float32)


def _heavy(dist_params):
    return pl.pallas_call(
        _heavy_body,
        grid=(NRB_H + 1,),
        in_specs=[pl.BlockSpec((RB, NCAT),
                               lambda s: (jnp.minimum(s, NRB_H - 1), 0))],
        out_specs=pl.BlockSpec((RB, NCAT), lambda s: (jnp.maximum(s - 1, 0), 0)),
        out_shape=jax.ShapeDtypeStruct((BATCH, NCAT), jnp.float32),
        scratch_shapes=[pltpu.VMEM((TC_ROWS, 1), jnp.int32)],
    )(dist_params)


def _light_body(x_ref, bits_ref, part_ref, out_ref, idx_scr):
    s = pl.program_id(0)
    del part_ref

    @pl.when(s < NRB_L)
    def _argmax():
        col0 = lax.broadcasted_iota(jnp.int32, (RB, CWL), 1)

        def body(j, carry):
            run_z, run_c = carry
            col = j * CWL + col0
            bits = lax.bitcast_convert_type(bits_ref[:, pl.ds(j * CWL, CWL)],
                                            jnp.uint32)
            x = x_ref[:, pl.ds(j * CWL, CWL)]
            z = x + _gumbel_from_bits(bits)
            better = z > run_z
            return (jnp.where(better, z, run_z), jnp.where(better, col, run_c))

        init = (jnp.full((RB, CWL), -jnp.inf, jnp.float32),
                jnp.zeros((RB, CWL), jnp.int32))
        run_z, run_c = lax.fori_loop(0, NFULL_L, body, init)

        colt = NFULL_L * CWL + lax.broadcasted_iota(jnp.int32, (RB, TAIL_L), 1)
        bt = lax.bitcast_convert_type(bits_ref[:, pl.ds(NFULL_L * CWL, TAIL_L)],
                                      jnp.uint32)
        xt = x_ref[:, pl.ds(NFULL_L * CWL, TAIL_L)]
        zt = xt + _gumbel_from_bits(bt)

        big = jnp.int32(2**31 - 1)
        rmax = jnp.maximum(jnp.max(run_z, axis=1, keepdims=True),
                           jnp.max(zt, axis=1, keepdims=True))
        cand = jnp.min(jnp.where(run_z == rmax, run_c, big),
                       axis=1, keepdims=True)
        candt = jnp.min(jnp.where(zt == rmax, colt, big),
                        axis=1, keepdims=True)
        idx_scr[pl.ds(s * RB, RB), :] = jnp.minimum(cand, candt)

    @pl.when(s > 0)
    def _onehot():
        idx = idx_scr[pl.ds((s - 1) * RB, RB), :]
        col = lax.broadcasted_iota(jnp.int32, (RB, NCAT), 1)
        out_ref[...] = (col == idx).astype(jnp.float32)


def _light(dist_params, bits, partial):
    return pl.pallas_call(
        _light_body,
        grid=(NRB_L + 1,),
        in_specs=[
            pl.BlockSpec((RB, NCAT),
                         lambda s: (NRB_H + jnp.minimum(s, NRB_L - 1), 0)),
            pl.BlockSpec((RB, NCAT), lambda s: (jnp.minimum(s, NRB_L - 1), 0)),
            pl.BlockSpec(memory_space=pl.ANY),
        ],
        out_specs=pl.BlockSpec((RB, NCAT),
                               lambda s: (NRB_H + jnp.maximum(s - 1, 0), 0)),
        out_shape=jax.ShapeDtypeStruct((BATCH, NCAT), jnp.float32),
        scratch_shapes=[pltpu.VMEM((SC_ROWS, 1), jnp.int32)],
        input_output_aliases={2: 0},
    )(dist_params, bits, partial)


@jax.jit
def kernel(dist_params):
    bits = _sc_bits()
    partial = _heavy(dist_params)
    return _light(dist_params, bits, partial)


# speedup vs baseline: 1.2524x; 1.2524x over previous
import jax
import jax.numpy as jnp
from jax import lax
from jax.experimental import pallas as pl
from jax.experimental.pallas import tpu as pltpu
from jax.experimental.pallas import tpu_sc as plsc

BATCH = 128
NCAT = 100000
RB = 8

SC_ROWS = 24
TC_ROWS = BATCH - SC_ROWS
NRB_H = TC_ROWS // RB
NRB_L = SC_ROWS // RB

CW = 2048
NFULL_H = NCAT // CW
TAIL_H = NCAT - NFULL_H * CW

CWL = 1024
NFULL_L = NCAT // CWL
TAIL_L = NCAT - NFULL_L * CWL

NWORK = 32
CHK = 1600
NCHUNK = SC_ROWS * NCAT // CHK
NITER = -(-NCHUNK // NWORK)
VPG = 10

_K2 = 1234
_KS2 = _K2 ^ 0x1BD11BDA
_ROT0 = (13, 15, 26, 6)
_ROT1 = (17, 29, 16, 24)


def _rotl(x, d):
    return (x << jnp.uint32(d)) | (x >> jnp.uint32(32 - d))


def _threefry_bits(x1):
    x0 = x1
    x1 = _rotl(x1, 13) ^ x0
    for r in _ROT0[1:]:
        x0 = x0 + x1
        x1 = _rotl(x1, r) ^ x0
    x0 = x0 + jnp.uint32(_K2)
    x1 = x1 + jnp.uint32(_KS2 + 1)
    for r in _ROT1:
        x0 = x0 + x1
        x1 = _rotl(x1, r) ^ x0
    x0 = x0 + jnp.uint32(_KS2)
    x1 = x1 + jnp.uint32(2)
    for r in _ROT0:
        x0 = x0 + x1
        x1 = _rotl(x1, r) ^ x0
    x1 = x1 + jnp.uint32(_K2 + 3)
    for r in _ROT1:
        x0 = x0 + x1
        x1 = _rotl(x1, r) ^ x0
    x0 = x0 + jnp.uint32(_K2)
    x1 = x1 + jnp.uint32(_KS2 + 4)
    for r in _ROT0:
        x0 = x0 + x1
        x1 = _rotl(x1, r) ^ x0
    x0 = x0 + jnp.uint32(_KS2)
    x1 = x1 + jnp.uint32(5)
    return x0 ^ x1


def _gumbel_from_bits(bits):
    fb = (bits >> jnp.uint32(9)) | jnp.uint32(0x3F800000)
    floats = lax.bitcast_convert_type(fb, jnp.float32) - jnp.float32(1.0)
    u = jnp.maximum(jnp.float32(1.1754943508222875e-38), floats)
    return -jnp.log(-jnp.log(u))


_SC_CNT_BASE = TC_ROWS * NCAT + _K2


def _sc_bits_body(out_hbm, buf, lane):
    wid = lax.axis_index("s") * 2 + lax.axis_index("c")
    lane[...] = lax.iota(jnp.int32, 16)

    def chunk(t, _):
        c = wid + t * NWORK

        @pl.when(c < NCHUNK)
        def _():
            cnt0 = _SC_CNT_BASE + c * CHK

            def group(i, _):
                for u in range(VPG):
                    off = i * (VPG * 16) + u * 16
                    x1 = (cnt0 + off + lane[...]).astype(jnp.uint32)
                    buf[pl.ds(off, 16)] = lax.bitcast_convert_type(
                        _threefry_bits(x1), jnp.int32)
                return 0

            lax.fori_loop(0, CHK // (VPG * 16), group, 0)
            pltpu.sync_copy(buf, out_hbm.at[pl.ds(c * CHK, CHK)])

        return 0

    lax.fori_loop(0, NITER, chunk, 0)


def _sc_bits():
    kern = pl.kernel(
        _sc_bits_body,
        out_type=jax.ShapeDtypeStruct((SC_ROWS * NCAT,), jnp.int32),
        mesh=plsc.VectorSubcoreMesh(core_axis_name="c", subcore_axis_name="s"),
        scratch_types=[
            pltpu.VMEM((CHK,), jnp.int32),
            pltpu.VMEM((16,), jnp.int32),
        ],
    )
    return kern().reshape(SC_ROWS, NCAT)


def _heavy_body(x_ref, out_ref, idx_scr):
    s = pl.program_id(0)

    @pl.when(s < NRB_H)
    def _argmax():
        row = s * RB + lax.broadcasted_iota(jnp.int32, (RB, CW), 0)
        basep = row * NCAT + jnp.int32(_K2)
        col0 = lax.broadcasted_iota(jnp.int32, (RB, CW), 1)

        def body(j, carry):
            run_z, run_c = carry
            col = j * CW + col0
            x1 = (basep + col).astype(jnp.uint32)
            x = x_ref[:, pl.ds(j * CW, CW)]
            z = x + _gumbel_from_bits(_threefry_bits(x1))
            better = z > run_z
            return (jnp.where(better, z, run_z), jnp.where(better, col, run_c))

        init = (jnp.full((RB, CW), -jnp.inf, jnp.float32),
                jnp.zeros((RB, CW), jnp.int32))
        run_z, run_c = lax.fori_loop(0, NFULL_H, body, init)

        colt = NFULL_H * CW + lax.broadcasted_iota(jnp.int32, (RB, TAIL_H), 1)
        rowt = s * RB + lax.broadcasted_iota(jnp.int32, (RB, TAIL_H), 0)
        x1t = (rowt * NCAT + jnp.int32(_K2) + colt).astype(jnp.uint32)
        xt = x_ref[:, pl.ds(NFULL_H * CW, TAIL_H)]
        zt = xt + _gumbel_from_bits(_threefry_bits(x1t))

        big = jnp.int32(2**31 - 1)
        rmax = jnp.maximum(jnp.max(run_z, axis=1, keepdims=True),
                           jnp.max(zt, axis=1, keepdims=True))
        cand = jnp.min(jnp.where(run_z == rmax, run_c, big),
                       axis=1, keepdims=True)
        candt = jnp.min(jnp.where(zt == rmax, colt, big),
                        axis=1, keepdims=True)
        idx_scr[pl.ds(s * RB, RB), :] = jnp.minimum(cand, candt)

    @pl.when(s > 0)
    def _onehot():
        idx = idx_scr[pl.ds((s - 1) * RB, RB), :]
        col = lax.broadcasted_iota(jnp.int32, (RB, NCAT), 1)
        out_ref[...] = (col == idx).astype(jnp.float32)


def _heavy(dist_params):
    return pl.pallas_call(
        _heavy_body,
        grid=(NRB_H + 1,),
        in_specs=[pl.BlockSpec((RB, NCAT),
                               lambda s: (jnp.minimum(s, NRB_H - 1), 0))],
        out_specs=pl.BlockSpec((RB, NCAT), lambda s: (jnp.maximum(s - 1, 0), 0)),
        out_shape=jax.ShapeDtypeStruct((TC_ROWS, NCAT), jnp.float32),
        scratch_shapes=[pltpu.VMEM((TC_ROWS, 1), jnp.int32)],
    )(dist_params)


def _light_body(x_ref, bits_ref, out_ref, idx_scr):
    s = pl.program_id(0)

    @pl.when(s < NRB_L)
    def _argmax():
        col0 = lax.broadcasted_iota(jnp.int32, (RB, CWL), 1)

        def body(j, carry):
            run_z, run_c = carry
            col = j * CWL + col0
            bits = lax.bitcast_convert_type(bits_ref[:, pl.ds(j * CWL, CWL)],
                                            jnp.uint32)
            x = x_ref[:, pl.ds(j * CWL, CWL)]
            z = x + _gumbel_from_bits(bits)
            better = z > run_z
            return (jnp.where(better, z, run_z), jnp.where(better, col, run_c))

        init = (jnp.full((RB, CWL), -jnp.inf, jnp.float32),
                jnp.zeros((RB, CWL), jnp.int32))
        run_z, run_c = lax.fori_loop(0, NFULL_L, body, init)

        colt = NFULL_L * CWL + lax.broadcasted_iota(jnp.int32, (RB, TAIL_L), 1)
        bt = lax.bitcast_convert_type(bits_ref[:, pl.ds(NFULL_L * CWL, TAIL_L)],
                                      jnp.uint32)
        xt = x_ref[:, pl.ds(NFULL_L * CWL, TAIL_L)]
        zt = xt + _gumbel_from_bits(bt)

        big = jnp.int32(2**31 - 1)
        rmax = jnp.maximum(jnp.max(run_z, axis=1, keepdims=True),
                           jnp.max(zt, axis=1, keepdims=True))
        cand = jnp.min(jnp.where(run_z == rmax, run_c, big),
                       axis=1, keepdims=True)
        candt = jnp.min(jnp.where(zt == rmax, colt, big),
                        axis=1, keepdims=True)
        idx_scr[pl.ds(s * RB, RB), :] = jnp.minimum(cand, candt)

    @pl.when(s > 0)
    def _onehot():
        idx = idx_scr[pl.ds((s - 1) * RB, RB), :]
        col = lax.broadcasted_iota(jnp.int32, (RB, NCAT), 1)
        out_ref[...] = (col == idx).astype(jnp.float32)


def _light(dist_params, bits):
    return pl.pallas_call(
        _light_body,
        grid=(NRB_L + 1,),
        in_specs=[
            pl.BlockSpec((RB, NCAT),
                         lambda s: (NRB_H + jnp.minimum(s, NRB_L - 1), 0)),
            pl.BlockSpec((RB, NCAT), lambda s: (jnp.minimum(s, NRB_L - 1), 0)),
        ],
        out_specs=pl.BlockSpec((RB, NCAT), lambda s: (jnp.maximum(s - 1, 0), 0)),
        out_shape=jax.ShapeDtypeStruct((SC_ROWS, NCAT), jnp.float32),
        scratch_shapes=[pltpu.VMEM((SC_ROWS, 1), jnp.int32)],
    )(dist_params, bits)


@jax.jit
def kernel(dist_params):
    bits = _sc_bits()
    top = _heavy(dist_params)
    bot = _light(dist_params, bits)
    return jnp.concatenate([top, bot], axis=0)


# speedup vs baseline: 1.2917x; 1.0314x over previous
import jax
import jax.numpy as jnp
from jax import lax
from jax.experimental import pallas as pl
from jax.experimental.pallas import tpu as pltpu
from jax.experimental.pallas import tpu_sc as plsc

BATCH = 128
NCAT = 100000
RB = 8

SC_ROWS = 24
TC_ROWS = BATCH - SC_ROWS
NRB_H = TC_ROWS // RB
NRB_L = SC_ROWS // RB

CW = 2048
NFULL_H = NCAT // CW
TAIL_H = NCAT - NFULL_H * CW

CWL = 1024
NFULL_L = NCAT // CWL
TAIL_L = NCAT - NFULL_L * CWL

NWORK = 32
CHK = 1600
NCHUNK = SC_ROWS * NCAT // CHK
NITER = -(-NCHUNK // NWORK)
VPG = 10

_K2 = 1234
_KS2 = _K2 ^ 0x1BD11BDA
_ROT0 = (13, 15, 26, 6)
_ROT1 = (17, 29, 16, 24)


def _rotl(x, d):
    return (x << jnp.uint32(d)) | (x >> jnp.uint32(32 - d))


def _threefry_bits(x1):
    x0 = x1
    x1 = _rotl(x1, 13) ^ x0
    for r in _ROT0[1:]:
        x0 = x0 + x1
        x1 = _rotl(x1, r) ^ x0
    x0 = x0 + jnp.uint32(_K2)
    x1 = x1 + jnp.uint32(_KS2 + 1)
    for r in _ROT1:
        x0 = x0 + x1
        x1 = _rotl(x1, r) ^ x0
    x0 = x0 + jnp.uint32(_KS2)
    x1 = x1 + jnp.uint32(2)
    for r in _ROT0:
        x0 = x0 + x1
        x1 = _rotl(x1, r) ^ x0
    x1 = x1 + jnp.uint32(_K2 + 3)
    for r in _ROT1:
        x0 = x0 + x1
        x1 = _rotl(x1, r) ^ x0
    x0 = x0 + jnp.uint32(_K2)
    x1 = x1 + jnp.uint32(_KS2 + 4)
    for r in _ROT0:
        x0 = x0 + x1
        x1 = _rotl(x1, r) ^ x0
    x0 = x0 + jnp.uint32(_KS2)
    x1 = x1 + jnp.uint32(5)
    return x0 ^ x1


def _gumbel_from_bits(bits):
    fb = (bits >> jnp.uint32(9)) | jnp.uint32(0x3F800000)
    floats = lax.bitcast_convert_type(fb, jnp.float32) - jnp.float32(1.0)
    u = jnp.maximum(jnp.float32(1.1754943508222875e-38), floats)
    return -jnp.log(-jnp.log(u))


_SC_CNT_BASE = TC_ROWS * NCAT + _K2


def _sc_bits_body(out_hbm, buf, lane):
    wid = lax.axis_index("s") * 2 + lax.axis_index("c")
    lane[...] = lax.iota(jnp.int32, 16)

    def chunk(t, _):
        c = wid + t * NWORK

        @pl.when(c < NCHUNK)
        def _():
            cnt0 = _SC_CNT_BASE + c * CHK

            def group(i, _):
                for u in range(VPG):
                    off = i * (VPG * 16) + u * 16
                    x1 = (cnt0 + off + lane[...]).astype(jnp.uint32)
                    buf[pl.ds(off, 16)] = lax.bitcast_convert_type(
                        _threefry_bits(x1), jnp.int32)
                return 0

            lax.fori_loop(0, CHK // (VPG * 16), group, 0)
            pltpu.sync_copy(buf, out_hbm.at[pl.ds(c * CHK, CHK)])

        return 0

    lax.fori_loop(0, NITER, chunk, 0)


def _sc_bits():
    kern = pl.kernel(
        _sc_bits_body,
        out_type=jax.ShapeDtypeStruct((SC_ROWS * NCAT,), jnp.int32),
        mesh=plsc.VectorSubcoreMesh(core_axis_name="c", subcore_axis_name="s"),
        scratch_types=[
            pltpu.VMEM((CHK,), jnp.int32),
            pltpu.VMEM((16,), jnp.int32),
        ],
    )
    return kern().reshape(SC_ROWS, NCAT)


def _heavy_body(x_ref, idx_ref):
    s = pl.program_id(0)

    def _argmax():
        row = s * RB + lax.broadcasted_iota(jnp.int32, (RB, CW), 0)
        basep = row * NCAT + jnp.int32(_K2)
        col0 = lax.broadcasted_iota(jnp.int32, (RB, CW), 1)

        def body(j, carry):
            run_z, run_c = carry
            col = j * CW + col0
            x1 = (basep + col).astype(jnp.uint32)
            x = x_ref[:, pl.ds(j * CW, CW)]
            z = x + _gumbel_from_bits(_threefry_bits(x1))
            better = z > run_z
            return (jnp.where(better, z, run_z), jnp.where(better, col, run_c))

        init = (jnp.full((RB, CW), -jnp.inf, jnp.float32),
                jnp.zeros((RB, CW), jnp.int32))
        run_z, run_c = lax.fori_loop(0, NFULL_H, body, init)

        colt = NFULL_H * CW + lax.broadcasted_iota(jnp.int32, (RB, TAIL_H), 1)
        rowt = s * RB + lax.broadcasted_iota(jnp.int32, (RB, TAIL_H), 0)
        x1t = (rowt * NCAT + jnp.int32(_K2) + colt).astype(jnp.uint32)
        xt = x_ref[:, pl.ds(NFULL_H * CW, TAIL_H)]
        zt = xt + _gumbel_from_bits(_threefry_bits(x1t))

        big = jnp.int32(2**31 - 1)
        rmax = jnp.maximum(jnp.max(run_z, axis=1, keepdims=True),
                           jnp.max(zt, axis=1, keepdims=True))
        cand = jnp.min(jnp.where(run_z == rmax, run_c, big),
                       axis=1, keepdims=True)
        candt = jnp.min(jnp.where(zt == rmax, colt, big),
                        axis=1, keepdims=True)
        idx_ref[...] = jnp.minimum(cand, candt)

    _argmax()


def _heavy(dist_params):
    return pl.pallas_call(
        _heavy_body,
        grid=(NRB_H,),
        in_specs=[pl.BlockSpec((RB, NCAT), lambda s: (s, 0))],
        out_specs=pl.BlockSpec((RB, 1), lambda s: (s, 0)),
        out_shape=jax.ShapeDtypeStruct((TC_ROWS, 1), jnp.int32),
    )(dist_params)


def _light_body(x_ref, bits_ref, idxh_ref, out_ref, idx_scr):
    s = pl.program_id(0)

    @pl.when(s == 0)
    def _seed():
        idx_scr[pl.ds(0, TC_ROWS), :] = idxh_ref[...]

    @pl.when(s < NRB_L)
    def _argmax():
        col0 = lax.broadcasted_iota(jnp.int32, (RB, CWL), 1)

        def body(j, carry):
            run_z, run_c = carry
            col = j * CWL + col0
            bits = lax.bitcast_convert_type(bits_ref[:, pl.ds(j * CWL, CWL)],
                                            jnp.uint32)
            x = x_ref[:, pl.ds(j * CWL, CWL)]
            z = x + _gumbel_from_bits(bits)
            better = z > run_z
            return (jnp.where(better, z, run_z), jnp.where(better, col, run_c))

        init = (jnp.full((RB, CWL), -jnp.inf, jnp.float32),
                jnp.zeros((RB, CWL), jnp.int32))
        run_z, run_c = lax.fori_loop(0, NFULL_L, body, init)

        colt = NFULL_L * CWL + lax.broadcasted_iota(jnp.int32, (RB, TAIL_L), 1)
        bt = lax.bitcast_convert_type(bits_ref[:, pl.ds(NFULL_L * CWL, TAIL_L)],
                                      jnp.uint32)
        xt = x_ref[:, pl.ds(NFULL_L * CWL, TAIL_L)]
        zt = xt + _gumbel_from_bits(bt)

        big = jnp.int32(2**31 - 1)
        rmax = jnp.maximum(jnp.max(run_z, axis=1, keepdims=True),
                           jnp.max(zt, axis=1, keepdims=True))
        cand = jnp.min(jnp.where(run_z == rmax, run_c, big),
                       axis=1, keepdims=True)
        candt = jnp.min(jnp.where(zt == rmax, colt, big),
                        axis=1, keepdims=True)
        idx_scr[pl.ds(TC_ROWS + s * RB, RB), :] = jnp.minimum(cand, candt)

    @pl.when(s >= NRB_L)
    def _onehot():
        idx = idx_scr[pl.ds((s - NRB_L) * RB, RB), :]
        col = lax.broadcasted_iota(jnp.int32, (RB, NCAT), 1)
        out_ref[...] = (col == idx).astype(jnp.float32)


def _finish(x_bot, bits, idx_heavy):
    return pl.pallas_call(
        _light_body,
        grid=(NRB_L + BATCH // RB,),
        in_specs=[
            pl.BlockSpec((RB, NCAT), lambda s: (jnp.minimum(s, NRB_L - 1), 0)),
            pl.BlockSpec((RB, NCAT), lambda s: (jnp.minimum(s, NRB_L - 1), 0)),
            pl.BlockSpec((TC_ROWS, 1), lambda s: (0, 0)),
        ],
        out_specs=pl.BlockSpec((RB, NCAT),
                               lambda s: (jnp.maximum(s - NRB_L, 0), 0)),
        out_shape=jax.ShapeDtypeStruct((BATCH, NCAT), jnp.float32),
        scratch_shapes=[pltpu.VMEM((BATCH, 1), jnp.int32)],
    )(x_bot, bits, idx_heavy)


@jax.jit
def kernel(dist_params):
    bits = _sc_bits()
    idx_heavy = _heavy(dist_params)
    x_bot = lax.slice(dist_params, (TC_ROWS, 0), (BATCH, NCAT))
    return _finish(x_bot, bits, idx_heavy)


# speedup vs baseline: 1.3845x; 1.0718x over previous
import jax
import jax.numpy as jnp
from jax import lax
from jax.experimental import pallas as pl
from jax.experimental.pallas import tpu as pltpu

BATCH = 128
NCAT = 100000
RB = 8
NRB = BATCH // RB

CW = 1536
NFULL = NCAT // CW
TAIL = NCAT - NFULL * CW

_K2 = 1234
_KS2 = _K2 ^ 0x1BD11BDA
_ROT0 = (13, 15, 26, 6)
_ROT1 = (17, 29, 16, 24)


def _rotl(x, d):
    return (x << jnp.uint32(d)) | (x >> jnp.uint32(32 - d))


def _threefry_bits(x1):
    x0 = x1
    x1 = _rotl(x1, 13) ^ x0
    for r in _ROT0[1:]:
        x0 = x0 + x1
        x1 = _rotl(x1, r) ^ x0
    x0 = x0 + jnp.uint32(_K2)
    x1 = x1 + jnp.uint32(_KS2 + 1)
    for r in _ROT1:
        x0 = x0 + x1
        x1 = _rotl(x1, r) ^ x0
    x0 = x0 + jnp.uint32(_KS2)
    x1 = x1 + jnp.uint32(2)
    for r in _ROT0:
        x0 = x0 + x1
        x1 = _rotl(x1, r) ^ x0
    x1 = x1 + jnp.uint32(_K2 + 3)
    for r in _ROT1:
        x0 = x0 + x1
        x1 = _rotl(x1, r) ^ x0
    x0 = x0 + jnp.uint32(_K2)
    x1 = x1 + jnp.uint32(_KS2 + 4)
    for r in _ROT0:
        x0 = x0 + x1
        x1 = _rotl(x1, r) ^ x0
    x0 = x0 + jnp.uint32(_KS2)
    x1 = x1 + jnp.uint32(5)
    return x0 ^ x1


def _gumbel_from_bits(bits):
    fb = (bits >> jnp.uint32(9)) | jnp.uint32(0x3F800000)
    floats = lax.bitcast_convert_type(fb, jnp.float32) - jnp.float32(1.0)
    u = jnp.maximum(jnp.float32(1.1754943508222875e-38), floats)
    return -jnp.log(-jnp.log(u))


def _body(x_ref, out_ref, idx_scr):
    s = pl.program_id(0)

    @pl.when(s < NRB)
    def _argmax():
        row = s * RB + lax.broadcasted_iota(jnp.int32, (RB, CW), 0)
        basep = row * NCAT + jnp.int32(_K2)
        col0 = lax.broadcasted_iota(jnp.int32, (RB, CW), 1)

        def body(j, carry):
            run_z, run_c = carry
            col = j * CW + col0
            x1 = (basep + col).astype(jnp.uint32)
            x = x_ref[:, pl.ds(j * CW, CW)]
            z = x + _gumbel_from_bits(_threefry_bits(x1))
            better = z > run_z
            return (jnp.where(better, z, run_z), jnp.where(better, col, run_c))

        init = (jnp.full((RB, CW), -jnp.inf, jnp.float32),
                jnp.zeros((RB, CW), jnp.int32))
        run_z, run_c = lax.fori_loop(0, NFULL, body, init)

        colt = NFULL * CW + lax.broadcasted_iota(jnp.int32, (RB, TAIL), 1)
        rowt = s * RB + lax.broadcasted_iota(jnp.int32, (RB, TAIL), 0)
        x1t = (rowt * NCAT + jnp.int32(_K2) + colt).astype(jnp.uint32)
        xt = x_ref[:, pl.ds(NFULL * CW, TAIL)]
        zt = xt + _gumbel_from_bits(_threefry_bits(x1t))

        big = jnp.int32(2**31 - 1)
        rmax = jnp.maximum(jnp.max(run_z, axis=1, keepdims=True),
                           jnp.max(zt, axis=1, keepdims=True))
        cand = jnp.min(jnp.where(run_z == rmax, run_c, big),
                       axis=1, keepdims=True)
        candt = jnp.min(jnp.where(zt == rmax, colt, big),
                        axis=1, keepdims=True)
        idx_scr[pl.ds(s * RB, RB), :] = jnp.minimum(cand, candt)

    @pl.when(s > 0)
    def _onehot():
        idx = idx_scr[pl.ds((s - 1) * RB, RB), :]
        col = lax.broadcasted_iota(jnp.int32, (RB, NCAT), 1)
        out_ref[...] = (col == idx).astype(jnp.float32)


@jax.jit
def kernel(dist_params):
    return pl.pallas_call(
        _body,
        grid=(NRB + 1,),
        in_specs=[pl.BlockSpec((RB, NCAT), lambda s: (jnp.minimum(s, NRB - 1), 0))],
        out_specs=pl.BlockSpec((RB, NCAT), lambda s: (jnp.maximum(s - 1, 0), 0)),
        out_shape=jax.ShapeDtypeStruct((BATCH, NCAT), jnp.float32),
        scratch_shapes=[pltpu.VMEM((BATCH, 1), jnp.int32)],
    )(dist_params)
